# baseline (speedup 1.0000x reference)
import functools

import jax
import jax.numpy as jnp
from jax import lax
from jax.experimental import pallas as pl
from jax.experimental.pallas import tpu as pltpu
from jax.experimental.pallas import tpu_sc as plsc

_NC = 2
_NS = 16
_NW = _NC * _NS
_NSTREAMS = 4
_SHIFTS = (0, 11, 22)
_MASKS = (2047, 2047, 1023)
_NBINS = 2048
_SIGN = -2**31


def _make_row_sorter(R, N):
  assert R % _NW == 0 and N % 16 == 0
  rows_per_w = R // _NW
  nvec = N // 16
  mesh = plsc.VectorSubcoreMesh(core_axis_name="c", subcore_axis_name="s")

  @functools.partial(
      pl.kernel,
      out_type=jax.ShapeDtypeStruct((R, N), jnp.float32),
      mesh=mesh,
      scratch_types=[
          pltpu.VMEM((N,), jnp.float32),
          pltpu.VMEM((N,), jnp.float32),
          *([pltpu.VMEM((_NBINS,), jnp.int32)] * _NSTREAMS),
          pltpu.VMEM((_NBINS,), jnp.int32),
          pltpu.SMEM((_NBINS // 16,), jnp.int32),
      ],
      compiler_params=pltpu.CompilerParams(needs_layout_passes=False),
  )
  def sorter(x_hbm, out_hbm, buf_a, buf_b, *hists_and_scan):
    hists = hists_and_scan[:_NSTREAMS]
    tsum = hists_and_scan[_NSTREAMS]
    stot = hists_and_scan[_NSTREAMS + 1]
    qvec = nvec // _NSTREAMS
    wid = lax.axis_index("s") * _NC + lax.axis_index("c")
    zeros16 = jnp.zeros((16,), jnp.int32)
    ones16 = jnp.ones((16,), jnp.int32)

    def twiddle(v):
      m = lax.shift_right_arithmetic(v, 31)
      return v ^ (m | _SIGN)

    def untwiddle(t):
      m = lax.shift_right_arithmetic(t, 31)
      return t ^ (~m | _SIGN)

    def do_row(r, carry):
      row = wid * rows_per_w + r
      pltpu.sync_copy(x_hbm.at[row], buf_a)

      for p in range(3):
        src = buf_a if p % 2 == 0 else buf_b
        dst = buf_b if p % 2 == 0 else buf_a
        shift = _SHIFTS[p]
        maskv = _MASKS[p]
        first = p == 0
        final = p == 2

        def zero_hist(i, c):
          for hu in hists:
            hu[pl.ds(i * 16, 16)] = zeros16
          return c
        lax.fori_loop(0, _NBINS // 16, zero_hist, 0)

        def hist_vec(i, c, src=src, shift=shift, maskv=maskv, first=first):
          for u, hu in enumerate(hists):
            v = plsc.bitcast(src[pl.ds((u * qvec + i) * 16, 16)], jnp.int32)
            if first:
              v = twiddle(v)
            d = lax.shift_right_logical(v, shift) & maskv
            plsc.addupdate_scatter(hu, [d], ones16)
          return c
        lax.fori_loop(0, qvec, hist_vec, 0)

        def scan_local(i, c):
          sl = pl.ds(i * 16, 16)
          t = hists[0][sl]
          for hu in hists[1:]:
            t = t + hu[sl]
          tsum[sl] = plsc.cumsum(t) - t
          stot[i] = jnp.sum(t)
          return c
        lax.fori_loop(0, _NBINS // 16, scan_local, 0)

        def scan_tops(j, tot):
          s = stot[j]
          stot[j] = tot
          return tot + s
        lax.fori_loop(0, _NBINS // 16, scan_tops, jnp.int32(0))

        def scan_apply(i, c):
          sl = pl.ds(i * 16, 16)
          g = tsum[sl] + stot[i]
          acc = g
          for u, hu in enumerate(hists):
            h = hu[sl]
            hu[sl] = acc
            if u + 1 < _NSTREAMS:
              acc = acc + h
          return c
        lax.fori_loop(0, _NBINS // 16, scan_apply, 0)

        def place_vec(i, c, src=src, dst=dst, shift=shift, maskv=maskv,
                      first=first, final=final):
          for u, hu in enumerate(hists):
            v = plsc.bitcast(src[pl.ds((u * qvec + i) * 16, 16)], jnp.int32)
            if first:
              v = twiddle(v)
            d = lax.shift_right_logical(v, shift) & maskv
            occ, last = plsc.scan_count(d)
            base = plsc.load_gather(hu, [d])
            out_v = untwiddle(v) if final else v
            plsc.store_scatter(dst, [base + occ - 1],
                               plsc.bitcast(out_v, jnp.float32))
            plsc.addupdate_scatter(hu, [d], occ, mask=last)
          return c
        lax.fori_loop(0, qvec, place_vec, 0)

      pltpu.sync_copy(buf_b, out_hbm.at[row])
      return carry

    lax.fori_loop(0, rows_per_w, do_row, 0)

  return sorter


def _pair_mse_matrix(ss, st, chunk):
  K = ss.shape[1]
  assert K % chunk == 0
  nchunks = K // chunk

  def body(ss_ref, st_ref, out_ref):
    a = ss_ref[...]
    b = st_ref[...]
    d = a[:, None, :] - b[None, :, :]
    acc = jnp.sum(d * d, axis=-1)

    @pl.when(pl.program_id(0) == 0)
    def _():
      out_ref[...] = jnp.zeros_like(out_ref)

    out_ref[...] += acc

  return pl.pallas_call(
      body,
      grid=(nchunks,),
      in_specs=[
          pl.BlockSpec((4, chunk), lambda i: (0, i)),
          pl.BlockSpec((4, chunk), lambda i: (0, i)),
      ],
      out_specs=pl.BlockSpec((4, 4), lambda i: (0, 0)),
      out_shape=jax.ShapeDtypeStruct((4, 4), jnp.float32),
  )(ss, st)


def kernel(style_E_0_0, style_E_0_1, style_E_mask_0_0, style_E_mask_0_1,
           style_S_0_0, style_S_0_1, style_S_mask_0_0, style_S_mask_0_1,
           translate_E_0_0, translate_E_0_1, translate_E_mask_0_0,
           translate_E_mask_0_1, translate_S_0_0, translate_S_0_1,
           translate_S_mask_0_0, translate_S_mask_0_1, neg_idx):
  del style_E_mask_0_0, style_E_mask_0_1, style_S_mask_0_0, style_S_mask_0_1
  del translate_E_mask_0_0, translate_E_mask_0_1
  del translate_S_mask_0_0, translate_S_mask_0_1

  sort_big = _make_row_sorter(256, 16384)
  sort_small = _make_row_sorter(512, 4096)

  groups = []
  for style, trans, sorter, shp in (
      (style_E_0_0, translate_E_0_0, sort_big, (256, 16384)),
      (style_S_0_0, translate_S_0_0, sort_big, (256, 16384)),
      (style_E_0_1, translate_E_0_1, sort_small, (512, 4096)),
      (style_S_0_1, translate_S_0_1, sort_small, (512, 4096)),
  ):
    ss = sorter(style.reshape(shp))
    st = sorter(trans.reshape(shp))
    K = (shp[0] // 4) * shp[1]
    M = _pair_mse_matrix(ss.reshape(4, K), st.reshape(4, K), 16384)
    groups.append(M / jnp.float32(K))

  Mtot = groups[0] + groups[1] + groups[2] + groups[3]
  poss = jnp.diagonal(Mtot)
  cols = jnp.arange(4)
  neg = Mtot[neg_idx[:, 0], cols] + Mtot[neg_idx[:, 1], cols]
  return jnp.sum(poss / neg)


# speedup vs baseline: 1.0097x; 1.0097x over previous
import functools

import jax
import jax.numpy as jnp
from jax import lax
from jax.experimental import pallas as pl
from jax.experimental.pallas import tpu as pltpu
from jax.experimental.pallas import tpu_sc as plsc

_NC = 2
_NS = 16
_NW = _NC * _NS
_NSTREAMS = 4
_SHIFTS = (0, 11, 22)
_MASKS = (2047, 2047, 1023)
_NBINS = 2048
_SIGN = -2**31


def _make_row_sorter(R, N):
  assert R % _NW == 0 and N % 16 == 0
  rows_per_w = R // _NW
  nvec = N // 16
  mesh = plsc.VectorSubcoreMesh(core_axis_name="c", subcore_axis_name="s")

  @functools.partial(
      pl.kernel,
      out_type=jax.ShapeDtypeStruct((R, N), jnp.float32),
      mesh=mesh,
      scratch_types=[
          pltpu.VMEM((N,), jnp.float32),
          pltpu.VMEM((N,), jnp.float32),
          *([pltpu.VMEM((_NBINS,), jnp.int32)] * _NSTREAMS),
      ],
      compiler_params=pltpu.CompilerParams(needs_layout_passes=False),
  )
  def sorter(x_hbm, out_hbm, buf_a, buf_b, *hists):
    qvec = nvec // _NSTREAMS
    wid = lax.axis_index("s") * _NC + lax.axis_index("c")
    zeros16 = jnp.zeros((16,), jnp.int32)
    ones16 = jnp.ones((16,), jnp.int32)

    def twiddle(v):
      m = lax.shift_right_arithmetic(v, 31)
      return v ^ (m | _SIGN)

    def untwiddle(t):
      m = lax.shift_right_arithmetic(t, 31)
      return t ^ (~m | _SIGN)

    def do_row(r, carry):
      row = wid * rows_per_w + r
      pltpu.sync_copy(x_hbm.at[row], buf_a)

      for p in range(3):
        src = buf_a if p % 2 == 0 else buf_b
        dst = buf_b if p % 2 == 0 else buf_a
        shift = _SHIFTS[p]
        maskv = _MASKS[p]
        first = p == 0
        final = p == 2

        def zero_hist(i, c):
          for hu in hists:
            hu[pl.ds(i * 16, 16)] = zeros16
          return c
        lax.fori_loop(0, _NBINS // 16, zero_hist, 0)

        def hist_vec(i, c, src=src, shift=shift, maskv=maskv, first=first):
          for k in range(2):
            for u, hu in enumerate(hists):
              v = plsc.bitcast(
                  src[pl.ds((u * qvec + i * 2 + k) * 16, 16)], jnp.int32)
              if first:
                v = twiddle(v)
              d = lax.shift_right_logical(v, shift) & maskv
              plsc.addupdate_scatter(hu, [d], ones16)
          return c
        lax.fori_loop(0, qvec // 2, hist_vec, 0)

        def scan_hist(i, tot):
          sl = pl.ds(i * 16, 16)
          hs = [hu[sl] for hu in hists]
          t = hs[0]
          for hv in hs[1:]:
            t = t + hv
          g = plsc.cumsum(t) - t + tot
          acc = g
          for u, hu in enumerate(hists):
            hu[sl] = acc
            if u + 1 < _NSTREAMS:
              acc = acc + hs[u]
          return tot + jnp.sum(t)
        lax.fori_loop(0, _NBINS // 16, scan_hist, jnp.int32(0))

        def place_vec(i, c, src=src, dst=dst, shift=shift, maskv=maskv,
                      first=first, final=final):
          for k in range(2):
            for u, hu in enumerate(hists):
              v = plsc.bitcast(
                  src[pl.ds((u * qvec + i * 2 + k) * 16, 16)], jnp.int32)
              if first:
                v = twiddle(v)
              d = lax.shift_right_logical(v, shift) & maskv
              occ, last = plsc.scan_count(d)
              base = plsc.load_gather(hu, [d])
              out_v = untwiddle(v) if final else v
              plsc.store_scatter(dst, [base + occ - 1],
                                 plsc.bitcast(out_v, jnp.float32))
              plsc.addupdate_scatter(hu, [d], occ, mask=last)
          return c
        lax.fori_loop(0, qvec // 2, place_vec, 0)

      pltpu.sync_copy(buf_b, out_hbm.at[row])
      return carry

    lax.fori_loop(0, rows_per_w, do_row, 0)

  return sorter


def _pair_mse_matrix(ss, st, chunk):
  K = ss.shape[1]
  assert K % chunk == 0
  nchunks = K // chunk

  def body(ss_ref, st_ref, out_ref):
    a = ss_ref[...]
    b = st_ref[...]
    d = a[:, None, :] - b[None, :, :]
    acc = jnp.sum(d * d, axis=-1)

    @pl.when(pl.program_id(0) == 0)
    def _():
      out_ref[...] = jnp.zeros_like(out_ref)

    out_ref[...] += acc

  return pl.pallas_call(
      body,
      grid=(nchunks,),
      in_specs=[
          pl.BlockSpec((4, chunk), lambda i: (0, i)),
          pl.BlockSpec((4, chunk), lambda i: (0, i)),
      ],
      out_specs=pl.BlockSpec((4, 4), lambda i: (0, 0)),
      out_shape=jax.ShapeDtypeStruct((4, 4), jnp.float32),
  )(ss, st)


def kernel(style_E_0_0, style_E_0_1, style_E_mask_0_0, style_E_mask_0_1,
           style_S_0_0, style_S_0_1, style_S_mask_0_0, style_S_mask_0_1,
           translate_E_0_0, translate_E_0_1, translate_E_mask_0_0,
           translate_E_mask_0_1, translate_S_0_0, translate_S_0_1,
           translate_S_mask_0_0, translate_S_mask_0_1, neg_idx):
  del style_E_mask_0_0, style_E_mask_0_1, style_S_mask_0_0, style_S_mask_0_1
  del translate_E_mask_0_0, translate_E_mask_0_1
  del translate_S_mask_0_0, translate_S_mask_0_1

  sort_big = _make_row_sorter(256, 16384)
  sort_small = _make_row_sorter(512, 4096)

  groups = []
  for style, trans, sorter, shp in (
      (style_E_0_0, translate_E_0_0, sort_big, (256, 16384)),
      (style_S_0_0, translate_S_0_0, sort_big, (256, 16384)),
      (style_E_0_1, translate_E_0_1, sort_small, (512, 4096)),
      (style_S_0_1, translate_S_0_1, sort_small, (512, 4096)),
  ):
    ss = sorter(style.reshape(shp))
    st = sorter(trans.reshape(shp))
    K = (shp[0] // 4) * shp[1]
    M = _pair_mse_matrix(ss.reshape(4, K), st.reshape(4, K), 16384)
    groups.append(M / jnp.float32(K))

  Mtot = groups[0] + groups[1] + groups[2] + groups[3]
  poss = jnp.diagonal(Mtot)
  cols = jnp.arange(4)
  neg = Mtot[neg_idx[:, 0], cols] + Mtot[neg_idx[:, 1], cols]
  return jnp.sum(poss / neg)
